# baseline (speedup 1.0000x reference)
import functools

import jax
import jax.numpy as jnp
from jax import lax
from jax.experimental import pallas as pl
from jax.experimental.pallas import tpu as pltpu
from jax.experimental.pallas import tpu_sc as plsc

_NC = 2
_NS = 16
_NW = _NC * _NS
_K = 128
_D = 128
_H = _D // _NC


def _mm_kernel(x_ref, w_ref, o_ref):
    o_ref[...] = lax.dot_general(
        x_ref[...], w_ref[...], (((1,), (1,)), ((), ())),
        preferred_element_type=jnp.float32)


def _tc_matmul(x, W):
    return pl.pallas_call(
        _mm_kernel,
        out_shape=jax.ShapeDtypeStruct((x.shape[0], W.shape[0]), jnp.float32),
    )(x, W)


def _scale_kernel(dp_ref, xp_ref, y_ref, s_ref):
    deg = dp_ref[:, 0:1] + dp_ref[:, 1:2] + 1.0
    s = lax.rsqrt(deg)
    s_ref[...] = s
    y_ref[...] = s * xp_ref[...]


def _tc_scale(dp, xp):
    n = xp.shape[0]
    return pl.pallas_call(
        _scale_kernel,
        out_shape=(jax.ShapeDtypeStruct((n, _D), jnp.float32),
                   jax.ShapeDtypeStruct((n, 1), jnp.float32)),
    )(dp, xp)


def _mid_kernel(ap_ref, y1_ref, s_ref, b1_ref, g_ref, be_ref, w2_ref,
                y2_ref):
    s = s_ref[...]
    agg = jnp.concatenate([ap_ref[0], ap_ref[1]], axis=1)
    pre = s * agg + s * y1_ref[...] + b1_ref[...]
    mu = jnp.mean(pre, axis=0, keepdims=True)
    var = jnp.mean((pre - mu) ** 2, axis=0, keepdims=True)
    h = (pre - mu) * lax.rsqrt(var + 1e-5) * g_ref[...] + be_ref[...]
    h = jnp.maximum(h, 0.0)
    xp2 = lax.dot_general(h, w2_ref[...], (((1,), (1,)), ((), ())),
                          preferred_element_type=jnp.float32)
    y2_ref[...] = s * xp2


def _tc_mid(ap, y1, s, b1, gamma, beta, W2):
    n = s.shape[0]
    return pl.pallas_call(
        _mid_kernel,
        out_shape=jax.ShapeDtypeStruct((n, _D), jnp.float32),
    )(ap, y1, s, b1, gamma, beta, W2)


def _fin_kernel(ap_ref, y2_ref, s_ref, b2_ref, o_ref):
    s = s_ref[...]
    agg = jnp.concatenate([ap_ref[0], ap_ref[1]], axis=1)
    o_ref[...] = s * agg + s * y2_ref[...] + b2_ref[...]


def _tc_fin(ap, y2, s, b2):
    n = s.shape[0]
    return pl.pallas_call(
        _fin_kernel,
        out_shape=jax.ShapeDtypeStruct((n, _D), jnp.float32),
    )(ap, y2, s, b2)


def _sc_mesh():
    return plsc.VectorSubcoreMesh(core_axis_name="c", subcore_axis_name="s")


@functools.partial(jax.jit, static_argnames=("n_pad", "nb", "hb"))
def _sc_deg(col4, ew4, *, n_pad, nb, hb):
    rt = n_pad // _NS
    zr = 16
    ncp = rt // zr

    @functools.partial(
        pl.kernel,
        out_type=jax.ShapeDtypeStruct((_NC, n_pad, _D), jnp.float32),
        mesh=_sc_mesh(),
        scratch_types=[
            pltpu.VMEM((hb, _K), jnp.int32),
            pltpu.VMEM((hb, _K), jnp.float32),
            pltpu.VMEM((_K, _D), jnp.float32),
            pltpu.VMEM((zr, _D), jnp.float32),
            pltpu.VMEM_SHARED((n_pad, _D), jnp.float32),
        ],
    )
    def k(col_hbm, ew_hbm, out_hbm, colv, ewv, stg, zb, acc):
        c = lax.axis_index("c")
        t = lax.axis_index("s")
        wid = t * _NC + c

        @pl.loop(0, zr)
        def _(i):
            for kk in range(_D // 16):
                zb[i, pl.ds(kk * 16, 16)] = jnp.zeros((16,), jnp.float32)

        for q in range(ncp):
            pltpu.sync_copy(zb, acc.at[pl.ds(t * rt + q * zr, zr)])

        @pl.loop(0, _K)
        def _(i):
            for kk in range(_D // 16):
                stg[i, pl.ds(kk * 16, 16)] = jnp.zeros((16,), jnp.float32)

        plsc.subcore_barrier()

        for blk in range(nb):
            pltpu.sync_copy(col_hbm.at[wid, blk], colv)
            pltpu.sync_copy(ew_hbm.at[wid, blk], ewv)

            @pl.loop(0, hb)
            def _(j):
                @pl.loop(0, _K, step=16)
                def _(i16):
                    w16 = ewv[j, pl.ds(i16, 16)]
                    for l in range(16):
                        stg[i16 + l, pl.ds(0, 16)] = jnp.full(
                            (16,), w16[l], jnp.float32)
                pltpu.sync_copy(stg, acc.at[colv.at[j]], add=True)

        plsc.subcore_barrier()
        for q in range(ncp):
            r0 = t * rt + q * zr
            pltpu.sync_copy(acc.at[pl.ds(r0, zr)], out_hbm.at[c, pl.ds(r0, zr)])

    return k(col4, ew4)


@functools.partial(jax.jit, static_argnames=("n_pad2", "nb", "hb"))
def _sc_agg(row4, colp4, ewe4, ewo4, y, *, n_pad2, nb, hb):
    rt = n_pad2 // _NS
    zr = 16
    ncp = rt // zr
    pairs = hb // 2

    @functools.partial(
        pl.kernel,
        out_type=jax.ShapeDtypeStruct((_NC, n_pad2, _D), jnp.float32),
        mesh=_sc_mesh(),
        scratch_types=[
            pltpu.VMEM((hb, _K), jnp.int32),
            pltpu.VMEM((hb, _K), jnp.int32),
            pltpu.VMEM((hb, _K), jnp.float32),
            pltpu.VMEM((hb, _K), jnp.float32),
            pltpu.VMEM((_K, _D), jnp.float32),
            pltpu.VMEM((_K, _D), jnp.float32),
            pltpu.VMEM((_K, _D), jnp.float32),
            pltpu.VMEM((16, _D), jnp.float32),
            pltpu.VMEM_SHARED((n_pad2, _D), jnp.float32),
            pltpu.SemaphoreType.DMA,
            pltpu.SemaphoreType.DMA,
        ],
    )
    def k(row_hbm, colp_hbm, ewe_hbm, ewo_hbm, y_hbm, out_hbm,
          rowv, colv, ewe, ewo, ra, rb, stg, zb, acc, sem_a, sem_b):
        c = lax.axis_index("c")
        t = lax.axis_index("s")
        coff = c * _H

        @pl.loop(0, zr)
        def _(i):
            for kk in range(_D // 16):
                zb[i, pl.ds(kk * 16, 16)] = jnp.zeros((16,), jnp.float32)

        for q in range(ncp):
            pltpu.sync_copy(zb, acc.at[pl.ds(t * rt + q * zr, zr)])

        def process(buf, j):
            @plsc.parallel_loop(0, _K, step=16, unroll=4)
            def _(i16):
                we16 = ewe[j, pl.ds(i16, 16)]
                wo16 = ewo[j, pl.ds(i16, 16)]
                for l in range(16):
                    wev = jnp.full((16,), we16[l], jnp.float32)
                    wov = jnp.full((16,), wo16[l], jnp.float32)
                    for kk in range(_H // 16):
                        v = buf[i16 + l, pl.ds(coff + kk * 16, 16)]
                        stg[i16 + l, pl.ds(kk * 16, 16)] = v * wev
                        stg[i16 + l, pl.ds(_H + kk * 16, 16)] = v * wov
            pltpu.sync_copy(stg, acc.at[colv.at[j]], add=True)

        plsc.subcore_barrier()

        @pl.loop(0, nb)
        def _(blk):
            pltpu.sync_copy(row_hbm.at[t, blk], rowv)
            pltpu.sync_copy(colp_hbm.at[t, blk], colv)
            pltpu.sync_copy(ewe_hbm.at[t, blk], ewe)
            pltpu.sync_copy(ewo_hbm.at[t, blk], ewo)

            pltpu.async_copy(y_hbm.at[rowv.at[0]], ra, sem_a)

            @pl.loop(0, pairs)
            def _(g):
                j0 = 2 * g
                j1 = j0 + 1
                pltpu.make_async_copy(y_hbm.at[rowv.at[j0]], ra, sem_a).wait()
                pltpu.async_copy(y_hbm.at[rowv.at[j1]], rb, sem_b)
                process(ra, j0)
                pltpu.make_async_copy(y_hbm.at[rowv.at[j1]], rb, sem_b).wait()

                @pl.when(j1 + 1 < hb)
                def _():
                    pltpu.async_copy(y_hbm.at[rowv.at[j1 + 1]], ra, sem_a)

                process(rb, j1)

            if hb % 2 == 1:
                pltpu.make_async_copy(y_hbm.at[rowv.at[hb - 1]], ra,
                                      sem_a).wait()
                process(ra, hb - 1)

        plsc.subcore_barrier()
        for q in range(ncp):
            r0 = t * rt + q * zr
            pltpu.sync_copy(acc.at[pl.ds(r0, zr)], out_hbm.at[c, pl.ds(r0, zr)])

    return k(row4, colp4, ewe4, ewo4, y)


def _ceil_to(v, q):
    return (v + q - 1) // q * q


def kernel(x, edge_index, edge_weight, W1, b1, gamma, beta, W2, b2):
    n = x.shape[0]
    e = edge_index.shape[1]

    row = edge_index[0].astype(jnp.int32)
    col = edge_index[1].astype(jnp.int32)
    ew = edge_weight.astype(jnp.float32)

    def pad_to(a, ln, dt):
        if a.shape[0] == ln:
            return a
        return jnp.concatenate([a, jnp.zeros((ln - a.shape[0],), dt)])

    nb_d = 2
    e_deg = _ceil_to(e, _NW * _K * nb_d)
    hb_d = e_deg // (_NW * _K * nb_d)
    col4 = pad_to(col, e_deg, jnp.int32).reshape(_NW, nb_d, hb_d, _K)
    ew4 = pad_to(ew, e_deg, jnp.float32).reshape(_NW, nb_d, hb_d, _K)

    hb_a = 20
    e_agg = _ceil_to(e, _NS * _K * hb_a)
    nb_a = e_agg // (_NS * _K * hb_a)
    rowp = pad_to(row, e_agg, jnp.int32)
    colp = pad_to(col, e_agg, jnp.int32)
    ewp = pad_to(ew, e_agg, jnp.float32)
    par = (colp & 1).astype(jnp.float32)
    row4 = rowp.reshape(_NS, nb_a, hb_a, _K)
    colp4 = (colp >> 1).reshape(_NS, nb_a, hb_a, _K)
    ewe4 = (ewp * (1.0 - par)).reshape(_NS, nb_a, hb_a, _K)
    ewo4 = (ewp * par).reshape(_NS, nb_a, hb_a, _K)

    n_deg = _ceil_to(n, 256)
    n_pad2 = _ceil_to(_ceil_to(n, 2) // 2, 256)

    degp = _sc_deg(col4, ew4, n_pad=n_deg, nb=nb_d, hb=hb_d)
    xp1 = _tc_matmul(x, W1)

    dp = jnp.stack([degp[0, :n, 0], degp[1, :n, 0]], axis=1)
    y1, s = _tc_scale(dp, xp1)

    def unpack(aggp):
        return aggp.reshape(_NC, n_pad2 * 2, _H)[:, :n]

    agg1 = unpack(_sc_agg(row4, colp4, ewe4, ewo4, y1,
                          n_pad2=n_pad2, nb=nb_a, hb=hb_a))
    y2 = _tc_mid(agg1, y1, s, b1.reshape(1, _D), gamma.reshape(1, _D),
                 beta.reshape(1, _D), W2)
    agg2 = unpack(_sc_agg(row4, colp4, ewe4, ewo4, y2,
                          n_pad2=n_pad2, nb=nb_a, hb=hb_a))
    return _tc_fin(agg2, y2, s, b2.reshape(1, _D))


# speedup vs baseline: 1.0506x; 1.0506x over previous
import functools

import jax
import jax.numpy as jnp
from jax import lax
from jax.experimental import pallas as pl
from jax.experimental.pallas import tpu as pltpu
from jax.experimental.pallas import tpu_sc as plsc

_NC = 2
_NS = 16
_NW = _NC * _NS
_K = 128
_D = 128
_H = _D // _NC


def _mm_kernel(x_ref, w_ref, o_ref):
    o_ref[...] = lax.dot_general(
        x_ref[...], w_ref[...], (((1,), (1,)), ((), ())),
        preferred_element_type=jnp.float32)


def _tc_matmul(x, W):
    return pl.pallas_call(
        _mm_kernel,
        out_shape=jax.ShapeDtypeStruct((x.shape[0], W.shape[0]), jnp.float32),
    )(x, W)


def _scale_kernel(dp_ref, xp_ref, y_ref, s_ref):
    deg = dp_ref[:, 0:1] + dp_ref[:, 1:2] + 1.0
    s = lax.rsqrt(deg)
    s_ref[...] = s
    y_ref[...] = s * xp_ref[...]


def _tc_scale(dp, xp):
    n = xp.shape[0]
    return pl.pallas_call(
        _scale_kernel,
        out_shape=(jax.ShapeDtypeStruct((n, _D), jnp.float32),
                   jax.ShapeDtypeStruct((n, 1), jnp.float32)),
    )(dp, xp)


def _mid_kernel(ap_ref, y1_ref, s_ref, b1_ref, g_ref, be_ref, w2_ref,
                y2_ref):
    s = s_ref[...]
    agg = jnp.concatenate([ap_ref[0], ap_ref[1]], axis=1)
    pre = s * agg + s * y1_ref[...] + b1_ref[...]
    mu = jnp.mean(pre, axis=0, keepdims=True)
    var = jnp.mean((pre - mu) ** 2, axis=0, keepdims=True)
    h = (pre - mu) * lax.rsqrt(var + 1e-5) * g_ref[...] + be_ref[...]
    h = jnp.maximum(h, 0.0)
    xp2 = lax.dot_general(h, w2_ref[...], (((1,), (1,)), ((), ())),
                          preferred_element_type=jnp.float32)
    y2_ref[...] = s * xp2


def _tc_mid(ap, y1, s, b1, gamma, beta, W2):
    n = s.shape[0]
    return pl.pallas_call(
        _mid_kernel,
        out_shape=jax.ShapeDtypeStruct((n, _D), jnp.float32),
    )(ap, y1, s, b1, gamma, beta, W2)


def _fin_kernel(ap_ref, y2_ref, s_ref, b2_ref, o_ref):
    s = s_ref[...]
    agg = jnp.concatenate([ap_ref[0], ap_ref[1]], axis=1)
    o_ref[...] = s * agg + s * y2_ref[...] + b2_ref[...]


def _tc_fin(ap, y2, s, b2):
    n = s.shape[0]
    return pl.pallas_call(
        _fin_kernel,
        out_shape=jax.ShapeDtypeStruct((n, _D), jnp.float32),
    )(ap, y2, s, b2)


def _sc_mesh():
    return plsc.VectorSubcoreMesh(core_axis_name="c", subcore_axis_name="s")


@functools.partial(jax.jit, static_argnames=("n_pad", "nb", "hb"))
def _sc_deg(col4, ew4, *, n_pad, nb, hb):
    rt = n_pad // _NS
    zr = 16
    ncp = rt // zr

    @functools.partial(
        pl.kernel,
        out_type=jax.ShapeDtypeStruct((_NC, n_pad, _D), jnp.float32),
        mesh=_sc_mesh(),
        scratch_types=[
            pltpu.VMEM((hb, _K), jnp.int32),
            pltpu.VMEM((hb, _K), jnp.float32),
            pltpu.VMEM((_K, _D), jnp.float32),
            pltpu.VMEM((zr, _D), jnp.float32),
            pltpu.VMEM_SHARED((n_pad, _D), jnp.float32),
        ],
    )
    def k(col_hbm, ew_hbm, out_hbm, colv, ewv, stg, zb, acc):
        c = lax.axis_index("c")
        t = lax.axis_index("s")
        wid = t * _NC + c

        @pl.loop(0, zr)
        def _(i):
            for kk in range(_D // 16):
                zb[i, pl.ds(kk * 16, 16)] = jnp.zeros((16,), jnp.float32)

        for q in range(ncp):
            pltpu.sync_copy(zb, acc.at[pl.ds(t * rt + q * zr, zr)])

        @pl.loop(0, _K)
        def _(i):
            for kk in range(_D // 16):
                stg[i, pl.ds(kk * 16, 16)] = jnp.zeros((16,), jnp.float32)

        plsc.subcore_barrier()

        for blk in range(nb):
            pltpu.sync_copy(col_hbm.at[wid, blk], colv)
            pltpu.sync_copy(ew_hbm.at[wid, blk], ewv)

            @pl.loop(0, hb)
            def _(j):
                @pl.loop(0, _K, step=16)
                def _(i16):
                    w16 = ewv[j, pl.ds(i16, 16)]
                    for l in range(16):
                        stg[i16 + l, pl.ds(0, 16)] = jnp.full(
                            (16,), w16[l], jnp.float32)
                pltpu.sync_copy(stg, acc.at[colv.at[j]], add=True)

        plsc.subcore_barrier()
        for q in range(ncp):
            r0 = t * rt + q * zr
            pltpu.sync_copy(acc.at[pl.ds(r0, zr)], out_hbm.at[c, pl.ds(r0, zr)])

    return k(col4, ew4)


@functools.partial(jax.jit, static_argnames=("n_pad2", "nb", "hb"))
def _sc_agg(row4, colp4, ewe4, ewo4, y, *, n_pad2, nb, hb):
    rt = n_pad2 // _NS
    zr = 16
    ncp = rt // zr
    pairs = hb // 2

    @functools.partial(
        pl.kernel,
        out_type=jax.ShapeDtypeStruct((_NC, n_pad2, _D), jnp.float32),
        mesh=_sc_mesh(),
        scratch_types=[
            pltpu.VMEM((hb, _K), jnp.int32),
            pltpu.VMEM((hb, _K), jnp.int32),
            pltpu.VMEM((hb, _K), jnp.float32),
            pltpu.VMEM((hb, _K), jnp.float32),
            pltpu.VMEM((_K, _D), jnp.float32),
            pltpu.VMEM((_K, _D), jnp.float32),
            pltpu.VMEM((_K, _D), jnp.float32),
            pltpu.VMEM((_K, _D), jnp.float32),
            pltpu.VMEM((16, _D), jnp.float32),
            pltpu.VMEM_SHARED((n_pad2, _D), jnp.float32),
            pltpu.SemaphoreType.DMA,
            pltpu.SemaphoreType.DMA,
            pltpu.SemaphoreType.DMA,
            pltpu.SemaphoreType.DMA,
        ],
    )
    def k(row_hbm, colp_hbm, ewe_hbm, ewo_hbm, y_hbm, out_hbm,
          rowv, colv, ewe, ewo, ra, rb, stga, stgb, zb, acc,
          sem_a, sem_b, sem_sa, sem_sb):
        c = lax.axis_index("c")
        t = lax.axis_index("s")
        coff = c * _H

        @pl.loop(0, zr)
        def _(i):
            for kk in range(_D // 16):
                zb[i, pl.ds(kk * 16, 16)] = jnp.zeros((16,), jnp.float32)

        for q in range(ncp):
            pltpu.sync_copy(zb, acc.at[pl.ds(t * rt + q * zr, zr)])

        def compute(buf, stg, j):
            @plsc.parallel_loop(0, _K, step=16, unroll=2)
            def _(i16):
                we16 = ewe[j, pl.ds(i16, 16)]
                wo16 = ewo[j, pl.ds(i16, 16)]
                for l in range(16):
                    wev = jnp.full((16,), we16[l], jnp.float32)
                    wov = jnp.full((16,), wo16[l], jnp.float32)
                    for kk in range(_H // 16):
                        v = buf[i16 + l, pl.ds(coff + kk * 16, 16)]
                        stg[i16 + l, pl.ds(kk * 16, 16)] = v * wev
                        stg[i16 + l, pl.ds(_H + kk * 16, 16)] = v * wov

        plsc.subcore_barrier()

        @pl.loop(0, nb)
        def _(blk):
            pltpu.sync_copy(row_hbm.at[t, blk], rowv)
            pltpu.sync_copy(colp_hbm.at[t, blk], colv)
            pltpu.sync_copy(ewe_hbm.at[t, blk], ewe)
            pltpu.sync_copy(ewo_hbm.at[t, blk], ewo)

            pltpu.async_copy(y_hbm.at[rowv.at[0]], ra, sem_a)

            @pl.loop(0, pairs)
            def _(g):
                j0 = 2 * g
                j1 = j0 + 1
                pltpu.make_async_copy(y_hbm.at[rowv.at[j0]], ra, sem_a).wait()
                pltpu.async_copy(y_hbm.at[rowv.at[j1]], rb, sem_b)

                @pl.when(j0 > 0)
                def _():
                    pltpu.make_async_copy(
                        stga, acc.at[colv.at[j0 - 2]], sem_sa).wait()

                compute(ra, stga, j0)
                pltpu.async_copy(stga, acc.at[colv.at[j0]], sem_sa, add=True)
                pltpu.make_async_copy(y_hbm.at[rowv.at[j1]], rb, sem_b).wait()

                @pl.when(j1 + 1 < hb)
                def _():
                    pltpu.async_copy(y_hbm.at[rowv.at[j1 + 1]], ra, sem_a)

                @pl.when(j1 > 1)
                def _():
                    pltpu.make_async_copy(
                        stgb, acc.at[colv.at[j1 - 2]], sem_sb).wait()

                compute(rb, stgb, j1)
                pltpu.async_copy(stgb, acc.at[colv.at[j1]], sem_sb, add=True)

            pltpu.make_async_copy(stga, acc.at[colv.at[hb - 2]], sem_sa).wait()
            pltpu.make_async_copy(stgb, acc.at[colv.at[hb - 1]], sem_sb).wait()

        plsc.subcore_barrier()
        for q in range(ncp):
            r0 = t * rt + q * zr
            pltpu.sync_copy(acc.at[pl.ds(r0, zr)], out_hbm.at[c, pl.ds(r0, zr)])

    return k(row4, colp4, ewe4, ewo4, y)


def _ceil_to(v, q):
    return (v + q - 1) // q * q


def kernel(x, edge_index, edge_weight, W1, b1, gamma, beta, W2, b2):
    n = x.shape[0]
    e = edge_index.shape[1]

    row = edge_index[0].astype(jnp.int32)
    col = edge_index[1].astype(jnp.int32)
    ew = edge_weight.astype(jnp.float32)

    def pad_to(a, ln, dt):
        if a.shape[0] == ln:
            return a
        return jnp.concatenate([a, jnp.zeros((ln - a.shape[0],), dt)])

    nb_d = 2
    e_deg = _ceil_to(e, _NW * _K * nb_d)
    hb_d = e_deg // (_NW * _K * nb_d)
    col4 = pad_to(col, e_deg, jnp.int32).reshape(_NW, nb_d, hb_d, _K)
    ew4 = pad_to(ew, e_deg, jnp.float32).reshape(_NW, nb_d, hb_d, _K)

    hb_a = 8
    e_agg = _ceil_to(e, _NS * _K * hb_a)
    nb_a = e_agg // (_NS * _K * hb_a)
    rowp = pad_to(row, e_agg, jnp.int32)
    colp = pad_to(col, e_agg, jnp.int32)
    ewp = pad_to(ew, e_agg, jnp.float32)
    par = (colp & 1).astype(jnp.float32)
    row4 = rowp.reshape(_NS, nb_a, hb_a, _K)
    colp4 = (colp >> 1).reshape(_NS, nb_a, hb_a, _K)
    ewe4 = (ewp * (1.0 - par)).reshape(_NS, nb_a, hb_a, _K)
    ewo4 = (ewp * par).reshape(_NS, nb_a, hb_a, _K)

    n_deg = _ceil_to(n, 256)
    n_pad2 = _ceil_to(_ceil_to(n, 2) // 2, 256)

    degp = _sc_deg(col4, ew4, n_pad=n_deg, nb=nb_d, hb=hb_d)
    xp1 = _tc_matmul(x, W1)

    dp = jnp.stack([degp[0, :n, 0], degp[1, :n, 0]], axis=1)
    y1, s = _tc_scale(dp, xp1)

    def unpack(aggp):
        return aggp.reshape(_NC, n_pad2 * 2, _H)[:, :n]

    agg1 = unpack(_sc_agg(row4, colp4, ewe4, ewo4, y1,
                          n_pad2=n_pad2, nb=nb_a, hb=hb_a))
    y2 = _tc_mid(agg1, y1, s, b1.reshape(1, _D), gamma.reshape(1, _D),
                 beta.reshape(1, _D), W2)
    agg2 = unpack(_sc_agg(row4, colp4, ewe4, ewo4, y2,
                          n_pad2=n_pad2, nb=nb_a, hb=hb_a))
    return _tc_fin(agg2, y2, s, b2.reshape(1, _D))
